# baseline (device time: 7220 ns/iter reference)
import jax
import jax.numpy as jnp
from jax import lax
from jax.experimental import pallas as pl
from jax.experimental.pallas import tpu as pltpu


def kernel(x):
    m_per, n_per = x.shape

    def body(x_hbm, out_ref, x_vmem, send_buf, recv_buf,
             copy_sem, send_sem, recv_sem):
        my_x = lax.axis_index("x")
        my_y = lax.axis_index("y")
        peer = (my_x, 1 - my_y)

        in_copy = pltpu.make_async_copy(x_hbm, x_vmem, copy_sem)
        in_copy.start()

        barrier_sem = pltpu.get_barrier_semaphore()
        pl.semaphore_signal(
            barrier_sem, inc=1, device_id=peer,
            device_id_type=pl.DeviceIdType.MESH,
        )

        pl.semaphore_wait(barrier_sem, 1)

        in_copy.wait()
        send_buf[:, :] = jnp.sum(x_vmem[:, :], axis=1).reshape(12, 128)

        rdma = pltpu.make_async_remote_copy(
            src_ref=send_buf,
            dst_ref=recv_buf,
            send_sem=send_sem,
            recv_sem=recv_sem,
            device_id=peer,
            device_id_type=pl.DeviceIdType.MESH,
        )
        rdma.start()
        rdma.wait()

        out_ref[:, :] = send_buf[:, :] + recv_buf[:, :]

    out_packed = pl.pallas_call(
        body,
        out_shape=jax.ShapeDtypeStruct((12, 128), jnp.float32),
        in_specs=[pl.BlockSpec(memory_space=pl.MemorySpace.ANY)],
        out_specs=pl.BlockSpec(memory_space=pltpu.VMEM),
        scratch_shapes=[
            pltpu.VMEM((m_per, n_per), jnp.float32),
            pltpu.VMEM((12, 128), jnp.float32),
            pltpu.VMEM((12, 128), jnp.float32),
            pltpu.SemaphoreType.DMA,
            pltpu.SemaphoreType.DMA,
            pltpu.SemaphoreType.DMA,
        ],
        compiler_params=pltpu.CompilerParams(collective_id=0),
    )(x)
    return out_packed.reshape(m_per, 1)


# device time: 7195 ns/iter; 1.0035x vs baseline; 1.0035x over previous
import jax
import jax.numpy as jnp
from jax import lax
from jax.experimental import pallas as pl
from jax.experimental.pallas import tpu as pltpu


def kernel(x):
    m_per, n_per = x.shape

    def body(x_hbm, out_ref, x_vmem, send_buf, recv_buf,
             copy_sem, send_sem, recv_sem):
        my_x = lax.axis_index("x")
        my_y = lax.axis_index("y")
        peer = (my_x, 1 - my_y)

        in_copy = pltpu.make_async_copy(x_hbm, x_vmem, copy_sem)
        in_copy.start()

        barrier_sem = pltpu.get_barrier_semaphore()
        pl.semaphore_signal(
            barrier_sem, inc=1, device_id=peer,
            device_id_type=pl.DeviceIdType.MESH,
        )

        pl.semaphore_wait(barrier_sem, 1)

        in_copy.wait()
        send_buf[:, :] = jnp.sum(x_vmem[:, :], axis=1).reshape(12, 128)

        rdma = pltpu.make_async_remote_copy(
            src_ref=send_buf,
            dst_ref=recv_buf,
            send_sem=send_sem,
            recv_sem=recv_sem,
            device_id=peer,
            device_id_type=pl.DeviceIdType.MESH,
        )
        rdma.start()
        rdma.wait()

        out_ref[:, :] = send_buf[:, :] + recv_buf[:, :]

    out_packed = pl.pallas_call(
        body,
        out_shape=jax.ShapeDtypeStruct((12, 128), jnp.float32),
        in_specs=[pl.BlockSpec(memory_space=pltpu.MemorySpace.HBM)],
        out_specs=pl.BlockSpec(memory_space=pltpu.VMEM),
        scratch_shapes=[
            pltpu.VMEM((m_per, n_per), jnp.float32),
            pltpu.VMEM((12, 128), jnp.float32),
            pltpu.VMEM((12, 128), jnp.float32),
            pltpu.SemaphoreType.DMA,
            pltpu.SemaphoreType.DMA,
            pltpu.SemaphoreType.DMA,
        ],
        compiler_params=pltpu.CompilerParams(collective_id=0),
    )(x)
    return out_packed.reshape(m_per, 1)


# device time: 6873 ns/iter; 1.0505x vs baseline; 1.0468x over previous
import jax
import jax.numpy as jnp
from jax import lax
from jax.experimental import pallas as pl
from jax.experimental.pallas import tpu as pltpu


def kernel(x):
    m_per, n_per = x.shape

    def body(x_hbm, out_ref, x_vmem, send_buf, recv_buf,
             copy_sem, send_sem, recv_sem):
        my_x = lax.axis_index("x")
        my_y = lax.axis_index("y")
        peer = (my_x, 1 - my_y)

        in_copy = pltpu.make_async_copy(x_hbm, x_vmem, copy_sem)
        in_copy.start()

        barrier_sem = pltpu.get_barrier_semaphore()
        pl.semaphore_signal(
            barrier_sem, inc=1, device_id=peer,
            device_id_type=pl.DeviceIdType.MESH,
        )

        pl.semaphore_wait(barrier_sem, 1)

        in_copy.wait()
        send_buf[:, :] = jnp.sum(x_vmem[:, :], axis=1).reshape(12, 128)

        rdma = pltpu.make_async_remote_copy(
            src_ref=send_buf,
            dst_ref=recv_buf,
            send_sem=send_sem,
            recv_sem=recv_sem,
            device_id=peer,
            device_id_type=pl.DeviceIdType.MESH,
        )
        rdma.start()
        rdma.wait()

        out_ref[:, :] = send_buf[:, :] + recv_buf[:, :]

    out_packed = pl.pallas_call(
        body,
        out_shape=jax.ShapeDtypeStruct((12, 128), jnp.float32),
        in_specs=[pl.BlockSpec(memory_space=pltpu.MemorySpace.HBM)],
        out_specs=pl.BlockSpec(memory_space=pltpu.VMEM),
        scratch_shapes=[
            pltpu.VMEM((m_per, n_per), jnp.float32),
            pltpu.VMEM((12, 128), jnp.float32),
            pltpu.VMEM((12, 128), jnp.float32),
            pltpu.SemaphoreType.DMA,
            pltpu.SemaphoreType.DMA,
            pltpu.SemaphoreType.DMA,
        ],
        compiler_params=pltpu.CompilerParams(collective_id=0),
    )(pltpu.with_memory_space_constraint(x, pltpu.MemorySpace.HBM))
    return out_packed.reshape(m_per, 1)


# device time: 6762 ns/iter; 1.0677x vs baseline; 1.0164x over previous
import jax
import jax.numpy as jnp
from jax import lax
from jax.experimental import pallas as pl
from jax.experimental.pallas import tpu as pltpu


def kernel(x):
    m_per, n_per = x.shape
    half = m_per // 2

    def body(x_hbm, out_ref, x_vmem, send_buf, recv_buf,
             copy_sems, send_sems, recv_sems):
        my_x = lax.axis_index("x")
        my_y = lax.axis_index("y")
        peer = (my_x, 1 - my_y)

        copies = []
        for c in range(2):
            cp = pltpu.make_async_copy(
                x_hbm.at[pl.ds(c * half, half)],
                x_vmem.at[pl.ds(c * half, half)],
                copy_sems.at[c],
            )
            cp.start()
            copies.append(cp)

        barrier_sem = pltpu.get_barrier_semaphore()
        pl.semaphore_signal(
            barrier_sem, inc=1, device_id=peer,
            device_id_type=pl.DeviceIdType.MESH,
        )
        pl.semaphore_wait(barrier_sem, 1)

        rdmas = []
        for c in range(2):
            copies[c].wait()
            send_buf[c] = jnp.sum(
                x_vmem[pl.ds(c * half, half), :], axis=1
            ).reshape(6, 128)
            rdma = pltpu.make_async_remote_copy(
                src_ref=send_buf.at[c],
                dst_ref=recv_buf.at[c],
                send_sem=send_sems.at[c],
                recv_sem=recv_sems.at[c],
                device_id=peer,
                device_id_type=pl.DeviceIdType.MESH,
            )
            rdma.start()
            rdmas.append(rdma)

        for c in range(2):
            rdmas[c].wait()
        out_ref[pl.ds(0, 6), :] = send_buf[0] + recv_buf[0]
        out_ref[pl.ds(6, 6), :] = send_buf[1] + recv_buf[1]

    out_packed = pl.pallas_call(
        body,
        out_shape=jax.ShapeDtypeStruct((12, 128), jnp.float32),
        in_specs=[pl.BlockSpec(memory_space=pltpu.MemorySpace.HBM)],
        out_specs=pl.BlockSpec(memory_space=pltpu.VMEM),
        scratch_shapes=[
            pltpu.VMEM((m_per, n_per), jnp.float32),
            pltpu.VMEM((2, 6, 128), jnp.float32),
            pltpu.VMEM((2, 6, 128), jnp.float32),
            pltpu.SemaphoreType.DMA((2,)),
            pltpu.SemaphoreType.DMA((2,)),
            pltpu.SemaphoreType.DMA((2,)),
        ],
        compiler_params=pltpu.CompilerParams(collective_id=0),
    )(pltpu.with_memory_space_constraint(x, pltpu.MemorySpace.HBM))
    return out_packed.reshape(m_per, 1)
